# Initial kernel scaffold; baseline (speedup 1.0000x reference)
import jax
import jax.numpy as jnp
from jax.experimental import pallas as pl


def kernel(x, Wg, We_gate, We_up, We_down, Ws_gate, Ws_up, Ws_down):
    raise NotImplementedError("write your pallas kernel here")


# speedup vs baseline: 1.4102x; 1.4102x over previous
import functools

import jax
import jax.numpy as jnp
from jax.experimental import pallas as pl
from jax.experimental.pallas import tpu as pltpu

B, S, H = 1, 2048, 1024
E, K, I = 8, 2, 512
T = B * S
EP = 16
TT = 512
NT = T // TT


def _router_kernel(x_ref, wg_ref, w16_ref):
    logits = jnp.dot(x_ref[...], wg_ref[...].T,
                     preferred_element_type=jnp.float32)
    m1 = jnp.max(logits, axis=-1, keepdims=True)
    masked = jnp.where(logits == m1, -jnp.inf, logits)
    m2 = jnp.max(masked, axis=-1, keepdims=True)
    sel = logits >= m2
    e = jnp.where(sel, jnp.exp(logits - m1), 0.0)
    w = e / jnp.sum(e, axis=-1, keepdims=True)
    w16 = jnp.pad(w, ((0, 0), (0, EP - E)))
    col = jax.lax.broadcasted_iota(jnp.int32, (T, EP), 1)
    w16_ref[...] = jnp.where(col == E, 1.0, w16)


def _moe_kernel(x_ref, w16_ref, wg_ref, wu_ref, wd_ref, o_ref, acc_ref):
    e = pl.program_id(0)
    t = pl.program_id(1)
    xb = x_ref[...]
    g = jnp.dot(xb, wg_ref[0].T, preferred_element_type=jnp.float32)
    u = jnp.dot(xb, wu_ref[0].T, preferred_element_type=jnp.float32)
    hmid = (g * jax.nn.sigmoid(g)) * u
    o = jnp.dot(hmid.astype(jnp.bfloat16), wd_ref[0].T,
                preferred_element_type=jnp.float32)
    col = jax.lax.broadcasted_iota(jnp.int32, (TT, EP), 1)
    wcol = jnp.sum(jnp.where(col == e, w16_ref[...], 0.0), axis=1,
                   keepdims=True)
    contrib = o * wcol

    @pl.when(e == 0)
    def _init():
        acc_ref[pl.ds(t * TT, TT), :] = contrib

    @pl.when(e > 0)
    def _acc():
        acc_ref[pl.ds(t * TT, TT), :] += contrib

    @pl.when(e == E)
    def _flush():
        o_ref[...] = acc_ref[pl.ds(t * TT, TT), :]


@jax.jit
def kernel(x, Wg, We_gate, We_up, We_down, Ws_gate, Ws_up, Ws_down):
    xf = x.reshape(T, H)
    w16 = pl.pallas_call(
        _router_kernel,
        out_shape=jax.ShapeDtypeStruct((T, EP), jnp.float32),
    )(xf, Wg)

    wcat_g = jnp.concatenate([We_gate, Ws_gate[None]], 0).astype(jnp.bfloat16)
    wcat_u = jnp.concatenate([We_up, Ws_up[None]], 0).astype(jnp.bfloat16)
    wcat_d = jnp.concatenate([We_down, Ws_down[None]], 0).astype(jnp.bfloat16)
    xbf = xf.astype(jnp.bfloat16)

    y = pl.pallas_call(
        _moe_kernel,
        grid=(E + 1, NT),
        in_specs=[
            pl.BlockSpec((TT, H), lambda e, t: (t, 0)),
            pl.BlockSpec((TT, EP), lambda e, t: (t, 0)),
            pl.BlockSpec((1, I, H), lambda e, t: (e, 0, 0)),
            pl.BlockSpec((1, I, H), lambda e, t: (e, 0, 0)),
            pl.BlockSpec((1, H, I), lambda e, t: (e, 0, 0)),
        ],
        out_specs=pl.BlockSpec((TT, H), lambda e, t: (t, 0)),
        out_shape=jax.ShapeDtypeStruct((T, H), jnp.float32),
        scratch_shapes=[pltpu.VMEM((T, H), jnp.float32)],
        compiler_params=pltpu.CompilerParams(
            dimension_semantics=("arbitrary", "arbitrary"),
        ),
    )(xbf, w16, wcat_g, wcat_u, wcat_d)
    return y.reshape(B, S, H)


# speedup vs baseline: 1.5543x; 1.1022x over previous
import functools

import jax
import jax.numpy as jnp
from jax.experimental import pallas as pl
from jax.experimental.pallas import tpu as pltpu

B, S, H = 1, 2048, 1024
E, K, I = 8, 2, 512
T = B * S
EP = 16
TT = 2048
NT = T // TT


def _router_kernel(x_ref, wg_ref, w16_ref):
    logits = jnp.dot(x_ref[...], wg_ref[...].T,
                     preferred_element_type=jnp.float32)
    m1 = jnp.max(logits, axis=-1, keepdims=True)
    masked = jnp.where(logits == m1, -jnp.inf, logits)
    m2 = jnp.max(masked, axis=-1, keepdims=True)
    sel = logits >= m2
    e = jnp.where(sel, jnp.exp(logits - m1), 0.0)
    w = e / jnp.sum(e, axis=-1, keepdims=True)
    w16 = jnp.pad(w, ((0, 0), (0, EP - E)))
    col = jax.lax.broadcasted_iota(jnp.int32, (T, EP), 1)
    w16_ref[...] = jnp.where(col == E, 1.0, w16)


def _moe_kernel(x_ref, w16_ref, wg_ref, wu_ref, wd_ref, o_ref):
    e = pl.program_id(0)
    xb = x_ref[...]
    g = jnp.dot(xb, wg_ref[0].T, preferred_element_type=jnp.float32)
    u = jnp.dot(xb, wu_ref[0].T, preferred_element_type=jnp.float32)
    hmid = (g * jax.nn.sigmoid(g)) * u
    o = jnp.dot(hmid.astype(jnp.bfloat16), wd_ref[0].T,
                preferred_element_type=jnp.float32)
    col = jax.lax.broadcasted_iota(jnp.int32, (TT, EP), 1)
    wcol = jnp.sum(jnp.where(col == e, w16_ref[...], 0.0), axis=1,
                   keepdims=True)
    contrib = o * wcol

    @pl.when(e == 0)
    def _init():
        o_ref[...] = contrib

    @pl.when(e > 0)
    def _acc():
        o_ref[...] += contrib


@jax.jit
def kernel(x, Wg, We_gate, We_up, We_down, Ws_gate, Ws_up, Ws_down):
    xf = x.reshape(T, H)
    w16 = pl.pallas_call(
        _router_kernel,
        out_shape=jax.ShapeDtypeStruct((T, EP), jnp.float32),
    )(xf, Wg)

    wcat_g = jnp.concatenate([We_gate, Ws_gate[None]], 0).astype(jnp.bfloat16)
    wcat_u = jnp.concatenate([We_up, Ws_up[None]], 0).astype(jnp.bfloat16)
    wcat_d = jnp.concatenate([We_down, Ws_down[None]], 0).astype(jnp.bfloat16)
    xbf = xf.astype(jnp.bfloat16)

    y = pl.pallas_call(
        _moe_kernel,
        grid=(E + 1,),
        in_specs=[
            pl.BlockSpec((TT, H), lambda e: (0, 0)),
            pl.BlockSpec((TT, EP), lambda e: (0, 0)),
            pl.BlockSpec((1, I, H), lambda e: (e, 0, 0)),
            pl.BlockSpec((1, I, H), lambda e: (e, 0, 0)),
            pl.BlockSpec((1, H, I), lambda e: (e, 0, 0)),
        ],
        out_specs=pl.BlockSpec((TT, H), lambda e: (0, 0)),
        out_shape=jax.ShapeDtypeStruct((T, H), jnp.float32),
        compiler_params=pltpu.CompilerParams(
            dimension_semantics=("arbitrary",),
        ),
    )(xbf, w16, wcat_g, wcat_u, wcat_d)
    return y.reshape(B, S, H)
